# Initial kernel scaffold; baseline (speedup 1.0000x reference)
import jax
import jax.numpy as jnp
from jax.experimental import pallas as pl


def kernel(WO, label, table):
    raise NotImplementedError("write your pallas kernel here")


# speedup vs baseline: 8.4814x; 8.4814x over previous
import jax
import jax.numpy as jnp
from jax.experimental import pallas as pl

_MARGIN = 1.0


def _loss_kernel(wo_ref, lab_ref, tabT_ref, out_ref):
    B = wo_ref.shape[0]
    C = tabT_ref.shape[1]
    wo = wo_ref[:]
    x2 = jnp.sum(wo * wo, axis=1, keepdims=True)
    nrm = jnp.sqrt(x2)
    wn = wo / jnp.maximum(nrm, 1e-12)
    xn2 = jnp.sum(wn * wn, axis=1, keepdims=True)

    tabT = tabT_ref[:]
    t2 = jnp.sum(tabT * tabT, axis=0, keepdims=True)
    dots = jnp.dot(wn, tabT, preferred_element_type=jnp.float32)
    d2 = (xn2 + t2) - 2.0 * dots

    lab = lab_ref[:]
    cols = jax.lax.broadcasted_iota(jnp.int32, (B, C), 1)
    is_lab = cols == lab
    lab_d2 = jnp.sum(jnp.where(is_lab, d2, 0.0), axis=1, keepdims=True)
    min_d2 = jnp.min(jnp.where(is_lab, jnp.inf, d2), axis=1, keepdims=True)
    lab_d = jnp.sqrt(jnp.maximum(lab_d2, 0.0))
    min_d = jnp.sqrt(jnp.maximum(min_d2, 0.0))
    s = jnp.sum(lab_d - min_d, axis=0, keepdims=True)
    out_ref[:, :] = _MARGIN + s / B


def kernel(WO, label, table):
    B, _ = WO.shape
    out = pl.pallas_call(
        _loss_kernel,
        out_shape=jax.ShapeDtypeStruct((1, 1), jnp.float32),
    )(WO, label.astype(jnp.int32).reshape(B, 1), table.T)
    return out[0, 0]
